# Initial kernel scaffold; baseline (speedup 1.0000x reference)
import jax
import jax.numpy as jnp
from jax.experimental import pallas as pl


def kernel(boxes, scores):
    raise NotImplementedError("write your pallas kernel here")


# speedup vs baseline: 34.6509x; 34.6509x over previous
import jax
import jax.numpy as jnp
from jax.experimental import pallas as pl
from jax.experimental.pallas import tpu as pltpu

_SCORE_THRESH = 0.05
_NMS_THRESH = 0.5
_B = 256


def _iou(x1a, y1a, x2a, y2a, x1b, y1b, x2b, y2b):
    xx1 = jnp.maximum(x1a, x1b)
    yy1 = jnp.maximum(y1a, y1b)
    xx2 = jnp.minimum(x2a, x2b)
    yy2 = jnp.minimum(y2a, y2b)
    inter = jnp.maximum(xx2 - xx1, 0.0) * jnp.maximum(yy2 - yy1, 0.0)
    area_a = (x2a - x1a) * (y2a - y1a)
    area_b = (x2b - x1b) * (y2b - y1b)
    union = area_a + area_b - inter
    return inter / jnp.maximum(union, 1e-12)


def _nms_kernel(x1r, y1r, x2r, y2r, vr, keep, mrow):
    nb = x1r.shape[0]
    B = x1r.shape[1]
    keep[...] = vr[...]

    def block_body(b, carry):
        keep_b0 = keep[pl.ds(b, 1), :]

        @pl.when(jnp.sum(keep_b0) > 0.0)
        def _():
            r1 = x1r[pl.ds(b, 1), :]
            r2 = y1r[pl.ds(b, 1), :]
            r3 = x2r[pl.ds(b, 1), :]
            r4 = y2r[pl.ds(b, 1), :]
            c1 = jnp.swapaxes(r1, 0, 1)
            c2 = jnp.swapaxes(r2, 0, 1)
            c3 = jnp.swapaxes(r3, 0, 1)
            c4 = jnp.swapaxes(r4, 0, 1)

            iou = _iou(c1, c2, c3, c4, r1, r2, r3, r4)
            ii = jax.lax.broadcasted_iota(jnp.int32, (B, B), 0)
            jj = jax.lax.broadcasted_iota(jnp.int32, (B, B), 1)
            mrow[...] = ((iou > _NMS_THRESH) & (jj > ii)).astype(jnp.float32)

            lane = jax.lax.broadcasted_iota(jnp.int32, (1, B), 1)

            def row_body(i, kp):
                row = mrow[pl.ds(i, 1), :]
                onehot = (lane == i).astype(jnp.float32)
                ki = jnp.max(kp * onehot)
                return kp * (1.0 - row * ki)

            kb = jax.lax.fori_loop(0, B, row_body, keep_b0)
            keep[pl.ds(b, 1), :] = kb
            kw = jnp.broadcast_to(kb, (8, B)).astype(jnp.bfloat16)

            def push_body(c, carry2):
                keep_c = keep[pl.ds(c, 1), :]

                @pl.when(jnp.sum(keep_c) > 0.0)
                def _():
                    s1 = x1r[pl.ds(c, 1), :]
                    s2 = y1r[pl.ds(c, 1), :]
                    s3 = x2r[pl.ds(c, 1), :]
                    s4 = y2r[pl.ds(c, 1), :]
                    iou_bc = _iou(c1, c2, c3, c4, s1, s2, s3, s4)
                    ov = (iou_bc > _NMS_THRESH).astype(jnp.bfloat16)
                    sup = jnp.dot(kw, ov, preferred_element_type=jnp.float32)
                    alivef = (sup[0:1, :] < 0.5).astype(jnp.float32)
                    keep[pl.ds(c, 1), :] = keep_c * alivef

                return 0

            jax.lax.fori_loop(b + 1, nb, push_body, 0)

        return 0

    jax.lax.fori_loop(0, nb, block_body, 0)


def kernel(boxes, scores):
    n = scores.shape[0]
    valid = scores > _SCORE_THRESH
    ss = jnp.where(valid, scores, jnp.float32(-1e30))
    order = jnp.argsort(-ss)
    bs = boxes[order]
    vs = valid[order]
    nb = -(-n // _B)
    npad = nb * _B
    pad = npad - n

    def prep(col):
        return jnp.pad(col, (0, pad)).reshape(nb, _B)

    x1 = prep(bs[:, 0])
    y1 = prep(bs[:, 1])
    x2 = prep(bs[:, 2])
    y2 = prep(bs[:, 3])
    vf = prep(vs.astype(jnp.float32))

    keep = pl.pallas_call(
        _nms_kernel,
        out_shape=jax.ShapeDtypeStruct((nb, _B), jnp.float32),
        scratch_shapes=[pltpu.VMEM((_B, _B), jnp.float32)],
    )(x1, y1, x2, y2, vf)

    keep_s = keep.reshape(npad)[:n] > 0.5
    keep_orig = jnp.zeros(n, dtype=bool).at[order].set(keep_s)
    m = keep_orig.astype(boxes.dtype)
    return jnp.concatenate([boxes * m[:, None], (scores * m)[:, None]], axis=1)


# speedup vs baseline: 120.6301x; 3.4813x over previous
import jax
import jax.numpy as jnp
from jax.experimental import pallas as pl
from jax.experimental.pallas import tpu as pltpu

_SCORE_THRESH = 0.05
_NMS_THRESH = 0.5
_B = 256


def _overlap(x1a, y1a, x2a, y2a, aa, x1b, y1b, x2b, y2b, ab):
    xx1 = jnp.maximum(x1a, x1b)
    yy1 = jnp.maximum(y1a, y1b)
    xx2 = jnp.minimum(x2a, x2b)
    yy2 = jnp.minimum(y2a, y2b)
    inter = jnp.maximum(xx2 - xx1, 0.0) * jnp.maximum(yy2 - yy1, 0.0)
    union = aa + ab - inter
    iou = inter / jnp.maximum(union, 1e-12)
    return iou > _NMS_THRESH


def _nms_kernel(x1r, y1r, x2r, y2r, ar, vr, keep):
    nb = x1r.shape[0]
    B = x1r.shape[1]
    keep[...] = vr[...]

    def block_body(b, carry):
        keep_b0 = keep[pl.ds(b, 1), :]

        @pl.when(jnp.sum(keep_b0) > 0.0)
        def _():
            r1 = x1r[pl.ds(b, 1), :]
            r2 = y1r[pl.ds(b, 1), :]
            r3 = x2r[pl.ds(b, 1), :]
            r4 = y2r[pl.ds(b, 1), :]
            ra = ar[pl.ds(b, 1), :]
            c1 = jnp.swapaxes(r1, 0, 1)
            c2 = jnp.swapaxes(r2, 0, 1)
            c3 = jnp.swapaxes(r3, 0, 1)
            c4 = jnp.swapaxes(r4, 0, 1)
            ca = jnp.swapaxes(ra, 0, 1)

            ov_bb = _overlap(c1, c2, c3, c4, ca, r1, r2, r3, r4, ra)
            ii = jax.lax.broadcasted_iota(jnp.int32, (B, B), 0)
            jj = jax.lax.broadcasted_iota(jnp.int32, (B, B), 1)
            m_bb = (ov_bb & (jj > ii)).astype(jnp.bfloat16)

            def fp_cond(st):
                return st[1]

            def fp_body(st):
                k, _ = st
                kw = jnp.broadcast_to(k, (8, B)).astype(jnp.bfloat16)
                sup = jnp.dot(kw, m_bb, preferred_element_type=jnp.float32)
                knew = keep_b0 * (sup[0:1, :] < 0.5).astype(jnp.float32)
                return knew, jnp.sum(jnp.abs(knew - k)) > 0.0

            kb, _ = jax.lax.while_loop(fp_cond, fp_body, (keep_b0, True))
            keep[pl.ds(b, 1), :] = kb
            kwp = jnp.broadcast_to(kb, (8, B)).astype(jnp.bfloat16)

            def push_body(c, carry2):
                s1 = x1r[pl.ds(c, 1), :]
                s2 = y1r[pl.ds(c, 1), :]
                s3 = x2r[pl.ds(c, 1), :]
                s4 = y2r[pl.ds(c, 1), :]
                sa = ar[pl.ds(c, 1), :]
                ov = _overlap(c1, c2, c3, c4, ca, s1, s2, s3, s4, sa)
                sup = jnp.dot(kwp, ov.astype(jnp.bfloat16),
                              preferred_element_type=jnp.float32)
                alivef = (sup[0:1, :] < 0.5).astype(jnp.float32)
                keep[pl.ds(c, 1), :] = keep[pl.ds(c, 1), :] * alivef
                return 0

            jax.lax.fori_loop(b + 1, nb, push_body, 0)

        return 0

    jax.lax.fori_loop(0, nb, block_body, 0)


def kernel(boxes, scores):
    n = scores.shape[0]
    valid = scores > _SCORE_THRESH
    ss = jnp.where(valid, scores, jnp.float32(-1e30))
    order = jnp.argsort(-ss)
    bs = boxes[order]
    vs = valid[order]
    nb = -(-n // _B)
    npad = nb * _B
    pad = npad - n

    def prep(col):
        return jnp.pad(col, (0, pad)).reshape(nb, _B)

    x1 = prep(bs[:, 0])
    y1 = prep(bs[:, 1])
    x2 = prep(bs[:, 2])
    y2 = prep(bs[:, 3])
    ar = prep((bs[:, 2] - bs[:, 0]) * (bs[:, 3] - bs[:, 1]))
    vf = prep(vs.astype(jnp.float32))

    keep = pl.pallas_call(
        _nms_kernel,
        out_shape=jax.ShapeDtypeStruct((nb, _B), jnp.float32),
    )(x1, y1, x2, y2, ar, vf)

    keep_s = keep.reshape(npad)[:n] > 0.5
    keep_orig = jnp.zeros(n, dtype=bool).at[order].set(keep_s)
    m = keep_orig.astype(boxes.dtype)
    return jnp.concatenate([boxes * m[:, None], (scores * m)[:, None]], axis=1)


# speedup vs baseline: 146.7203x; 1.2163x over previous
import jax
import jax.numpy as jnp
from jax.experimental import pallas as pl
from jax.experimental.pallas import tpu as pltpu

_SCORE_THRESH = 0.05
_NMS_THRESH = 0.5
_B = 256


def _overlap(x1a, y1a, x2a, y2a, aa, x1b, y1b, x2b, y2b, ab):
    xx1 = jnp.maximum(x1a, x1b)
    yy1 = jnp.maximum(y1a, y1b)
    xx2 = jnp.minimum(x2a, x2b)
    yy2 = jnp.minimum(y2a, y2b)
    inter = jnp.maximum(xx2 - xx1, 0.0) * jnp.maximum(yy2 - yy1, 0.0)
    union = aa + ab - inter
    iou = inter / jnp.maximum(union, 1e-12)
    return iou > _NMS_THRESH


def _nms_kernel(x1r, y1r, x2r, y2r, ar, vr, keep):
    nb = x1r.shape[0]
    B = x1r.shape[1]
    keep[...] = vr[...]

    def block_body(b, carry):
        keep_b0 = keep[pl.ds(b, 1), :]

        @pl.when(jnp.sum(keep_b0) > 0.0)
        def _():
            r1 = x1r[pl.ds(b, 1), :]
            r2 = y1r[pl.ds(b, 1), :]
            r3 = x2r[pl.ds(b, 1), :]
            r4 = y2r[pl.ds(b, 1), :]
            ra = ar[pl.ds(b, 1), :]
            c1 = jnp.swapaxes(r1, 0, 1)
            c2 = jnp.swapaxes(r2, 0, 1)
            c3 = jnp.swapaxes(r3, 0, 1)
            c4 = jnp.swapaxes(r4, 0, 1)
            ca = jnp.swapaxes(ra, 0, 1)

            ov_bb = _overlap(c1, c2, c3, c4, ca, r1, r2, r3, r4, ra)
            ii = jax.lax.broadcasted_iota(jnp.int32, (B, B), 0)
            jj = jax.lax.broadcasted_iota(jnp.int32, (B, B), 1)
            m_bb = (ov_bb & (jj > ii)).astype(jnp.bfloat16)

            def fp_cond(st):
                return st[1]

            def fp_body(st):
                k, _ = st
                kw = jnp.broadcast_to(k, (8, B)).astype(jnp.bfloat16)
                sup = jnp.dot(kw, m_bb, preferred_element_type=jnp.float32)
                knew = keep_b0 * (sup[0:1, :] < 0.5).astype(jnp.float32)
                return knew, jnp.sum(jnp.abs(knew - k)) > 0.0

            kb, _ = jax.lax.while_loop(fp_cond, fp_body, (keep_b0, True))
            keep[pl.ds(b, 1), :] = kb
            kwp = jnp.broadcast_to(kb, (8, B)).astype(jnp.bfloat16)

            def push_one(c):
                s1 = x1r[pl.ds(c, 1), :]
                s2 = y1r[pl.ds(c, 1), :]
                s3 = x2r[pl.ds(c, 1), :]
                s4 = y2r[pl.ds(c, 1), :]
                sa = ar[pl.ds(c, 1), :]
                ov = _overlap(c1, c2, c3, c4, ca, s1, s2, s3, s4, sa)
                sup = jnp.dot(kwp, ov.astype(jnp.bfloat16),
                              preferred_element_type=jnp.float32)
                alivef = (sup[0:1, :] < 0.5).astype(jnp.float32)
                keep[pl.ds(c, 1), :] = keep[pl.ds(c, 1), :] * alivef

            def push_body(t, carry2):
                cA = b + 1 + 2 * t
                cB = jnp.minimum(cA + 1, nb - 1)
                push_one(cA)
                push_one(cB)
                return 0

            jax.lax.fori_loop(0, (nb - b) // 2, push_body, 0)

        return 0

    jax.lax.fori_loop(0, nb, block_body, 0)


def kernel(boxes, scores):
    n = scores.shape[0]
    valid = scores > _SCORE_THRESH
    ss = jnp.where(valid, scores, jnp.float32(-1e30))
    order = jnp.argsort(-ss)
    bs = boxes[order]
    vs = valid[order]
    nb = -(-n // _B)
    npad = nb * _B
    pad = npad - n

    def prep(col):
        return jnp.pad(col, (0, pad)).reshape(nb, _B)

    x1 = prep(bs[:, 0])
    y1 = prep(bs[:, 1])
    x2 = prep(bs[:, 2])
    y2 = prep(bs[:, 3])
    ar = prep((bs[:, 2] - bs[:, 0]) * (bs[:, 3] - bs[:, 1]))
    vf = prep(vs.astype(jnp.float32))

    keep = pl.pallas_call(
        _nms_kernel,
        out_shape=jax.ShapeDtypeStruct((nb, _B), jnp.float32),
    )(x1, y1, x2, y2, ar, vf)

    keep_s = keep.reshape(npad)[:n] > 0.5
    keep_orig = jnp.zeros(n, dtype=bool).at[order].set(keep_s)
    m = keep_orig.astype(boxes.dtype)
    return jnp.concatenate([boxes * m[:, None], (scores * m)[:, None]], axis=1)


# speedup vs baseline: 162.7816x; 1.1095x over previous
import jax
import jax.numpy as jnp
from jax.experimental import pallas as pl
from jax.experimental.pallas import tpu as pltpu

_SCORE_THRESH = 0.05
_NMS_THRESH = 0.5
_B = 256


def _overlap(x1a, y1a, x2a, y2a, aa, x1b, y1b, x2b, y2b, ab):
    xx1 = jnp.maximum(x1a, x1b)
    yy1 = jnp.maximum(y1a, y1b)
    xx2 = jnp.minimum(x2a, x2b)
    yy2 = jnp.minimum(y2a, y2b)
    inter = jnp.maximum(xx2 - xx1, 0.0) * jnp.maximum(yy2 - yy1, 0.0)
    union = aa + ab - inter
    iou = inter / jnp.maximum(union, 1e-12)
    return iou > _NMS_THRESH


def _nms_kernel(x1r, y1r, x2r, y2r, ar, vr, keep):
    nb = x1r.shape[0]
    B = x1r.shape[1]
    keep[...] = vr[...]

    def block_body(b, carry):
        keep_b0 = keep[pl.ds(b, 1), :]

        @pl.when(jnp.sum(keep_b0) > 0.0)
        def _():
            r1 = x1r[pl.ds(b, 1), :]
            r2 = y1r[pl.ds(b, 1), :]
            r3 = x2r[pl.ds(b, 1), :]
            r4 = y2r[pl.ds(b, 1), :]
            ra = ar[pl.ds(b, 1), :]
            c1 = jnp.swapaxes(r1, 0, 1)
            c2 = jnp.swapaxes(r2, 0, 1)
            c3 = jnp.swapaxes(r3, 0, 1)
            c4 = jnp.swapaxes(r4, 0, 1)
            ca = jnp.swapaxes(ra, 0, 1)

            ov_bb = _overlap(c1, c2, c3, c4, ca, r1, r2, r3, r4, ra)
            ii = jax.lax.broadcasted_iota(jnp.int32, (B, B), 0)
            jj = jax.lax.broadcasted_iota(jnp.int32, (B, B), 1)
            m_bb = (ov_bb & (jj > ii)).astype(jnp.bfloat16)

            def fp_cond(st):
                return st[1]

            def fp_body(st):
                k, _ = st
                kw = jnp.broadcast_to(k, (8, B)).astype(jnp.bfloat16)
                sup = jnp.dot(kw, m_bb, preferred_element_type=jnp.float32)
                knew = keep_b0 * (sup[0:1, :] < 0.5).astype(jnp.float32)
                return knew, jnp.sum(jnp.abs(knew - k)) > 0.0

            kb, _ = jax.lax.while_loop(fp_cond, fp_body, (keep_b0, True))
            keep[pl.ds(b, 1), :] = kb
            kwp = jnp.broadcast_to(kb, (8, B)).astype(jnp.bfloat16)

            def push_one(c):
                s1 = x1r[pl.ds(c, 1), :]
                s2 = y1r[pl.ds(c, 1), :]
                s3 = x2r[pl.ds(c, 1), :]
                s4 = y2r[pl.ds(c, 1), :]
                sa = ar[pl.ds(c, 1), :]
                ov = _overlap(c1, c2, c3, c4, ca, s1, s2, s3, s4, sa)
                sup = jnp.dot(kwp, ov.astype(jnp.bfloat16),
                              preferred_element_type=jnp.float32)
                alivef = (sup[0:1, :] < 0.5).astype(jnp.float32)
                keep[pl.ds(c, 1), :] = keep[pl.ds(c, 1), :] * alivef

            def push_body(t, carry2):
                cA = b + 1 + 4 * t
                push_one(cA)
                push_one(jnp.minimum(cA + 1, nb - 1))
                push_one(jnp.minimum(cA + 2, nb - 1))
                push_one(jnp.minimum(cA + 3, nb - 1))
                return 0

            jax.lax.fori_loop(0, (nb - b + 2) // 4, push_body, 0)

        return 0

    jax.lax.fori_loop(0, nb, block_body, 0)


def kernel(boxes, scores):
    n = scores.shape[0]
    valid = scores > _SCORE_THRESH
    ss = jnp.where(valid, scores, jnp.float32(-1e30))
    order = jnp.argsort(-ss)
    bs = boxes[order]
    vs = valid[order]
    nb = -(-n // _B)
    npad = nb * _B
    pad = npad - n

    def prep(col):
        return jnp.pad(col, (0, pad)).reshape(nb, _B)

    x1 = prep(bs[:, 0])
    y1 = prep(bs[:, 1])
    x2 = prep(bs[:, 2])
    y2 = prep(bs[:, 3])
    ar = prep((bs[:, 2] - bs[:, 0]) * (bs[:, 3] - bs[:, 1]))
    vf = prep(vs.astype(jnp.float32))

    keep = pl.pallas_call(
        _nms_kernel,
        out_shape=jax.ShapeDtypeStruct((nb, _B), jnp.float32),
    )(x1, y1, x2, y2, ar, vf)

    keep_s = keep.reshape(npad)[:n] > 0.5
    keep_orig = jnp.zeros(n, dtype=bool).at[order].set(keep_s)
    m = keep_orig.astype(boxes.dtype)
    return jnp.concatenate([boxes * m[:, None], (scores * m)[:, None]], axis=1)
